# baseline (device time: 60042 ns/iter reference)
import jax
import jax.numpy as jnp
from jax import lax
from jax.experimental import pallas as pl
from jax.experimental.pallas import tpu as pltpu

N_DEV = 8
N_EXP = 32
CAP = 102


def kernel(x, router_W, route_idx, expert_W):
    n_tok, d_model = x.shape
    e_loc, _, d_hid = expert_W.shape

    def body(x_ref, idx_ref, ew_ref, out_ref,
             wbuf, hbuf, wsR, wrR, wsL, wrL, hsend, hrecv):
        my = lax.axis_index("i")
        right = lax.rem(my + 1, N_DEV)
        left = lax.rem(my + N_DEV - 1, N_DEV)

        e_idx = idx_ref[:, :]
        exp_iota = lax.broadcasted_iota(jnp.int32, (n_tok, N_EXP), 1)
        onehot = (e_idx == exp_iota).astype(jnp.float32)
        hbuf[my, 0:1, :] = jnp.sum(onehot, axis=0, keepdims=True)

        wbuf[0] = ew_ref[...].astype(jnp.bfloat16)

        bsem = pltpu.get_barrier_semaphore()
        for nbr in (left, right):
            pl.semaphore_signal(bsem, inc=1, device_id=(nbr,),
                                device_id_type=pl.DeviceIdType.MESH)
        pl.semaphore_wait(bsem, 2)

        def rdma_cw(h):
            if h == 3:
                return pltpu.make_async_remote_copy(
                    src_ref=wbuf.at[3, pl.ds(0, 2)],
                    dst_ref=wbuf.at[4, pl.ds(0, 2)],
                    send_sem=wsR.at[h], recv_sem=wrR.at[h],
                    device_id=(right,), device_id_type=pl.DeviceIdType.MESH)
            return pltpu.make_async_remote_copy(
                src_ref=wbuf.at[h], dst_ref=wbuf.at[h + 1],
                send_sem=wsR.at[h], recv_sem=wrR.at[h],
                device_id=(right,), device_id_type=pl.DeviceIdType.MESH)

        def rdma_ccw(h):
            if h == 3:
                return pltpu.make_async_remote_copy(
                    src_ref=wbuf.at[7, pl.ds(2, 2)],
                    dst_ref=wbuf.at[4, pl.ds(2, 2)],
                    send_sem=wsL.at[h], recv_sem=wrL.at[h],
                    device_id=(left,), device_id_type=pl.DeviceIdType.MESH)
            return pltpu.make_async_remote_copy(
                src_ref=wbuf.at[0 if h == 0 else 4 + h],
                dst_ref=wbuf.at[5 + h],
                send_sem=wsL.at[h], recv_sem=wrL.at[h],
                device_id=(left,), device_id_type=pl.DeviceIdType.MESH)

        rR = rdma_cw(0)
        rL = rdma_ccw(0)
        rR.start()
        rL.start()

        hist_rds = []
        for d in range(1, N_DEV):
            tgt = lax.rem(my + d, N_DEV)
            rd = pltpu.make_async_remote_copy(
                src_ref=hbuf.at[my], dst_ref=hbuf.at[my],
                send_sem=hsend.at[d - 1], recv_sem=hrecv.at[d - 1],
                device_id=(tgt,), device_id_type=pl.DeviceIdType.MESH)
            rd.start()
            hist_rds.append(rd)

        row_i = lax.broadcasted_iota(jnp.int32, (n_tok, n_tok), 0)
        col_i = lax.broadcasted_iota(jnp.int32, (n_tok, n_tok), 1)
        ltri = (row_i > col_i).astype(jnp.float32)
        ranks = jnp.dot(ltri, onehot, preferred_element_type=jnp.float32)
        rank_tok = jnp.sum(ranks * onehot, axis=1, keepdims=True)

        for rd in hist_rds:
            rd.wait()

        hop_iota = lax.broadcasted_iota(jnp.int32, (N_DEV, 1), 0)
        wmask = (hop_iota < my).astype(jnp.float32)
        hvals = hbuf[:, 0, :]
        offs = jnp.sum(hvals * wmask, axis=0, keepdims=True)
        off_tok = jnp.sum(onehot * offs, axis=1, keepdims=True)
        accept = (rank_tok + off_tok) < float(CAP)

        xv = x_ref[...].astype(jnp.bfloat16)
        out_ref[...] = jnp.zeros((n_tok, d_hid), jnp.float32)

        def compute_slot(s):
            o = lax.rem(my - s + 2 * N_DEV, N_DEV) if s <= 4 else \
                lax.rem(my + (s - 4), N_DEV)
            acc = None
            for k in range(e_loc):
                e = o * e_loc + k
                m = jnp.logical_and(e_idx == e, accept).astype(jnp.bfloat16)
                d = jnp.dot(xv * m, wbuf[s, k],
                            preferred_element_type=jnp.float32)
                acc = d if acc is None else acc + d
            out_ref[...] += acc

        compute_slot(0)
        rR.wait()
        rL.wait()

        for h in (1, 2, 3):
            rR = rdma_cw(h)
            rL = rdma_ccw(h)
            rR.start()
            rL.start()
            compute_slot(h)
            compute_slot(4 + h)
            rR.wait()
            rL.wait()
        compute_slot(4)

    out_shape = jax.ShapeDtypeStruct((n_tok, d_hid), jnp.float32)
    return pl.pallas_call(
        body,
        out_shape=out_shape,
        in_specs=[pl.BlockSpec(memory_space=pltpu.VMEM)] * 3,
        out_specs=pl.BlockSpec(memory_space=pltpu.VMEM),
        scratch_shapes=[
            pltpu.VMEM((N_DEV, e_loc, d_model, d_hid), jnp.bfloat16),
            pltpu.VMEM((N_DEV, 1, N_EXP), jnp.float32),
            pltpu.SemaphoreType.DMA((4,)),
            pltpu.SemaphoreType.DMA((4,)),
            pltpu.SemaphoreType.DMA((4,)),
            pltpu.SemaphoreType.DMA((4,)),
            pltpu.SemaphoreType.DMA((N_DEV - 1,)),
            pltpu.SemaphoreType.DMA((N_DEV - 1,)),
        ],
        compiler_params=pltpu.CompilerParams(collective_id=0),
    )(x, route_idx, expert_W)


# device time: 54916 ns/iter; 1.0933x vs baseline; 1.0933x over previous
import jax
import jax.numpy as jnp
from jax import lax
from jax.experimental import pallas as pl
from jax.experimental.pallas import tpu as pltpu

N_DEV = 8
N_EXP = 32
CAP = 102


def kernel(x, router_W, route_idx, expert_W):
    n_tok, d_model = x.shape
    e_loc, _, d_hid = expert_W.shape

    def body(x_ref, idx_ref, ew_ref, out_ref,
             wbuf, hbuf, wsR, wrR, wsL, wrL, hsend, hrecv):
        my = lax.axis_index("i")
        right = lax.rem(my + 1, N_DEV)
        left = lax.rem(my + N_DEV - 1, N_DEV)

        e_idx = idx_ref[:, :]
        exp_iota = lax.broadcasted_iota(jnp.int32, (n_tok, N_EXP), 1)
        onehot = (e_idx == exp_iota).astype(jnp.float32)
        hbuf[my, 0:1, :] = jnp.sum(onehot, axis=0, keepdims=True)

        wbuf[0] = ew_ref[...].astype(jnp.bfloat16)

        bsem = pltpu.get_barrier_semaphore()
        for nbr in (left, right):
            pl.semaphore_signal(bsem, inc=1, device_id=(nbr,),
                                device_id_type=pl.DeviceIdType.MESH)
        pl.semaphore_wait(bsem, 2)

        def cw_chunks(h):
            return (0, 1) if h == 3 else (0, 1, 2, 3)

        def ccw_chunks(h):
            return (2, 3) if h == 3 else (0, 1, 2, 3)

        def rdma_cw(h, j):
            return pltpu.make_async_remote_copy(
                src_ref=wbuf.at[3 if h == 3 else h, pl.ds(j, 1)],
                dst_ref=wbuf.at[h + 1, pl.ds(j, 1)],
                send_sem=wsR.at[4 * h + j], recv_sem=wrR.at[4 * h + j],
                device_id=(right,), device_id_type=pl.DeviceIdType.MESH)

        def rdma_ccw(h, j):
            src_slot = 0 if h == 0 else (7 if h == 3 else 4 + h)
            dst_slot = 4 if h == 3 else 5 + h
            return pltpu.make_async_remote_copy(
                src_ref=wbuf.at[src_slot, pl.ds(j, 1)],
                dst_ref=wbuf.at[dst_slot, pl.ds(j, 1)],
                send_sem=wsL.at[4 * h + j], recv_sem=wrL.at[4 * h + j],
                device_id=(left,), device_id_type=pl.DeviceIdType.MESH)

        prev_cw = []
        prev_ccw = []
        for j in range(4):
            rR = rdma_cw(0, j)
            rL = rdma_ccw(0, j)
            rR.start()
            rL.start()
            prev_cw.append(rR)
            prev_ccw.append(rL)

        hist_rds = []
        for d in range(1, N_DEV):
            tgt = lax.rem(my + d, N_DEV)
            rd = pltpu.make_async_remote_copy(
                src_ref=hbuf.at[my], dst_ref=hbuf.at[my],
                send_sem=hsend.at[d - 1], recv_sem=hrecv.at[d - 1],
                device_id=(tgt,), device_id_type=pl.DeviceIdType.MESH)
            rd.start()
            hist_rds.append(rd)

        row_i = lax.broadcasted_iota(jnp.int32, (n_tok, n_tok), 0)
        col_i = lax.broadcasted_iota(jnp.int32, (n_tok, n_tok), 1)
        ltri = (row_i > col_i).astype(jnp.float32)
        ranks = jnp.dot(ltri, onehot, preferred_element_type=jnp.float32)
        rank_tok = jnp.sum(ranks * onehot, axis=1, keepdims=True)

        for rd in hist_rds:
            rd.wait()

        hop_iota = lax.broadcasted_iota(jnp.int32, (N_DEV, 1), 0)
        wmask = (hop_iota < my).astype(jnp.float32)
        hvals = hbuf[:, 0, :]
        offs = jnp.sum(hvals * wmask, axis=0, keepdims=True)
        off_tok = jnp.sum(onehot * offs, axis=1, keepdims=True)
        accept = (rank_tok + off_tok) < float(CAP)

        xv = x_ref[...].astype(jnp.bfloat16)
        out_ref[...] = jnp.zeros((n_tok, d_hid), jnp.float32)

        def compute_slot(s):
            o = lax.rem(my - s + 2 * N_DEV, N_DEV) if s <= 4 else \
                lax.rem(my + (s - 4), N_DEV)
            acc = None
            for k in range(e_loc):
                e = o * e_loc + k
                m = jnp.logical_and(e_idx == e, accept).astype(jnp.bfloat16)
                d = jnp.dot(xv * m, wbuf[s, k],
                            preferred_element_type=jnp.float32)
                acc = d if acc is None else acc + d
            out_ref[...] += acc

        compute_slot(0)

        for h in (1, 2, 3):
            next_cw = []
            next_ccw = []
            for j in range(4):
                prev_cw[j].wait()
                if j in cw_chunks(h):
                    rR = rdma_cw(h, j)
                    rR.start()
                    next_cw.append(rR)
                prev_ccw[j].wait()
                if j in ccw_chunks(h):
                    rL = rdma_ccw(h, j)
                    rL.start()
                    next_ccw.append(rL)
            prev_cw, prev_ccw = next_cw, next_ccw
            compute_slot(h)
            compute_slot(4 + h)
        for rd in prev_cw + prev_ccw:
            rd.wait()
        compute_slot(4)

    out_shape = jax.ShapeDtypeStruct((n_tok, d_hid), jnp.float32)
    return pl.pallas_call(
        body,
        out_shape=out_shape,
        in_specs=[pl.BlockSpec(memory_space=pltpu.VMEM)] * 3,
        out_specs=pl.BlockSpec(memory_space=pltpu.VMEM),
        scratch_shapes=[
            pltpu.VMEM((N_DEV, e_loc, d_model, d_hid), jnp.bfloat16),
            pltpu.VMEM((N_DEV, 1, N_EXP), jnp.float32),
            pltpu.SemaphoreType.DMA((16,)),
            pltpu.SemaphoreType.DMA((16,)),
            pltpu.SemaphoreType.DMA((16,)),
            pltpu.SemaphoreType.DMA((16,)),
            pltpu.SemaphoreType.DMA((N_DEV - 1,)),
            pltpu.SemaphoreType.DMA((N_DEV - 1,)),
        ],
        compiler_params=pltpu.CompilerParams(collective_id=0),
    )(x, route_idx, expert_W)
